# baseline (speedup 1.0000x reference)
import jax
import jax.numpy as jnp
from jax.experimental import pallas as pl
from jax.experimental.pallas import tpu as pltpu
from jax.experimental.pallas import tpu_sc as plsc

_K_BLOCK = 4000
_K_PAD = 4096
_CHUNKS = _K_PAD // 128
_ROW_GROUP = 16
_GATHER_WINDOW = 128
_NEG = -3.0e38


def _topk_body(q_ref, k_ref, v1o, v2o, v3o, i1o, i2o, i3o,
               qn, sim_ref, v1L, v2L, v3L, c1L, c2L, c3L):
    step = pl.program_id(0)
    nsteps = pl.num_programs(0)
    nq = q_ref.shape[0]

    @pl.when(step == 0)
    def _():
        q0 = q_ref[...]
        qn[...] = 1.0 / jnp.sqrt(jnp.sum(q0 * q0, axis=1, keepdims=True))
        neg = jnp.full(v1L.shape, _NEG, jnp.float32)
        zero = jnp.zeros(c1L.shape, jnp.int32)
        v1L[...] = neg
        v2L[...] = neg
        v3L[...] = neg
        c1L[...] = zero
        c2L[...] = zero
        c3L[...] = zero
        sim_ref[:, _K_BLOCK:_K_PAD] = jnp.full((nq, _K_PAD - _K_BLOCK),
                                               _NEG, jnp.float32)

    kb = k_ref[...]
    sq = kb * kb
    ones = jnp.ones((8, kb.shape[1]), jnp.float32)
    knsq = jax.lax.dot_general(ones, sq, (((1,), (1,)), ((), ())),
                               preferred_element_type=jnp.float32,
                               precision=jax.lax.Precision.HIGHEST)[0:1]
    rkn = 1.0 / (jnp.sqrt(knsq) + 1e-30)
    s = jax.lax.dot_general(q_ref[...], kb, (((1,), (1,)), ((), ())),
                            preferred_element_type=jnp.float32)
    sim_ref[:, 0:_K_BLOCK] = s * rkn

    base_chunk = step * _CHUNKS

    def rg_body(r, _):
        rows = pl.ds(r * _ROW_GROUP, _ROW_GROUP)
        carry = (v1L[rows, :], v2L[rows, :], v3L[rows, :],
                 c1L[rows, :], c2L[rows, :], c3L[rows, :])

        for j in range(_CHUNKS):
            a1, a2, a3, d1, d2, d3 = carry
            x = sim_ref[rows, j * 128:(j + 1) * 128]
            cb = jnp.zeros(x.shape, jnp.int32) + (base_chunk + j)
            g1 = x > a1
            g2 = x > a2
            g3 = x > a3
            n3 = jnp.where(g2, a2, jnp.where(g3, x, a3))
            e3 = jnp.where(g2, d2, jnp.where(g3, cb, d3))
            n2 = jnp.where(g1, a1, jnp.where(g2, x, a2))
            e2 = jnp.where(g1, d1, jnp.where(g2, cb, d2))
            n1 = jnp.where(g1, x, a1)
            e1 = jnp.where(g1, cb, d1)
            carry = (n1, n2, n3, e1, e2, e3)

        v1L[rows, :], v2L[rows, :], v3L[rows, :] = carry[0], carry[1], carry[2]
        c1L[rows, :], c2L[rows, :], c3L[rows, :] = carry[3], carry[4], carry[5]
        return 0

    jax.lax.fori_loop(0, nq // _ROW_GROUP, rg_body, 0)

    @pl.when(step == nsteps - 1)
    def _():
        lane = jax.lax.broadcasted_iota(jnp.int32, (nq, 128), 1)

        def cand_idx(c):
            return (c >> 5) * _K_BLOCK + (c & 31) * 128 + lane

        vals = jnp.concatenate([v1L[...], v2L[...], v3L[...]], axis=1)
        idxs = jnp.concatenate([cand_idx(c1L[...]), cand_idx(c2L[...]),
                                cand_idx(c3L[...])], axis=1)
        big = jnp.int32(2 ** 30)

        def extract(vals, idxs):
            m = jnp.max(vals, axis=1, keepdims=True)
            b = jnp.min(jnp.where(vals == m, idxs, big), axis=1, keepdims=True)
            return m, b, jnp.where(idxs == b, _NEG, vals)

        m1, b1, vals = extract(vals, idxs)
        m2, b2, vals = extract(vals, idxs)
        m3, b3, _ = extract(vals, idxs)
        rq = qn[...]
        v1o[...] = m1 * rq
        v2o[...] = m2 * rq
        v3o[...] = m3 * rq
        i1o[...] = b1
        i2o[...] = b2
        i3o[...] = b3


def _run_topk(q, k):
    nq = q.shape[0]
    grid = (k.shape[0] // _K_BLOCK,)
    return pl.pallas_call(
        _topk_body,
        grid=grid,
        in_specs=[
            pl.BlockSpec((nq, q.shape[1]), lambda i: (0, 0)),
            pl.BlockSpec((_K_BLOCK, k.shape[1]), lambda i: (i, 0)),
        ],
        out_specs=[pl.BlockSpec((nq, 1), lambda i: (0, 0))] * 6,
        out_shape=[jax.ShapeDtypeStruct((nq, 1), jnp.float32)] * 3
        + [jax.ShapeDtypeStruct((nq, 1), jnp.int32)] * 3,
        scratch_shapes=[pltpu.VMEM((nq, 1), jnp.float32),
                        pltpu.VMEM((nq, _K_PAD), jnp.float32)]
        + [pltpu.VMEM((nq, 128), jnp.float32)] * 3
        + [pltpu.VMEM((nq, 128), jnp.int32)] * 3,
    )(q, k)


def _sc_gather(table, idx):
    n_idx = idx.shape[1]
    mesh = plsc.VectorSubcoreMesh(core_axis_name="core",
                                  subcore_axis_name="subcore")

    @pl.kernel(out_type=jax.ShapeDtypeStruct((n_idx, table.shape[1]),
                                             table.dtype),
               mesh=mesh)
    def _gather_kernel(x_hbm, i_hbm, o_hbm):
        def body(i_vmem, o_vmem):
            pltpu.sync_copy(x_hbm.at[i_vmem.at[0]], o_vmem)

        pltpu.emit_pipeline(
            body,
            grid=(n_idx // _GATHER_WINDOW,),
            in_specs=[pl.BlockSpec((1, _GATHER_WINDOW),
                                   index_map=lambda i: (0, i))],
            out_specs=[pl.BlockSpec((_GATHER_WINDOW, table.shape[1]),
                                    index_map=lambda i: (i, 0))],
            core_axis_name="subcore",
            dimension_semantics=(pltpu.PARALLEL,),
        )(i_hbm, o_hbm)

    return _gather_kernel(table, idx)


def _finish_body(g_ref, w1, w2, w3, i1, i2, i3,
                 wfc_ref, bfc_ref, wom_ref, bom_ref, o_ref):
    n = o_ref.shape[0]
    d = o_ref.shape[1]

    def _half(g, idx):
        par = (idx[...] % 2) == 1
        return jnp.where(par, g[:, d:2 * d], g[:, 0:d])

    g0 = _half(g_ref[0:n], i1)
    g1 = _half(g_ref[n:2 * n], i2)
    g2 = _half(g_ref[2 * n:3 * n], i3)
    a, b, c = w1[...], w2[...], w3[...]
    agg = (g0 * a + g1 * b + g2 * c) / (a + b + c)
    t = jax.lax.dot_general(agg, wfc_ref[...], (((1,), (1,)), ((), ())),
                            preferred_element_type=jnp.float32) + bfc_ref[...]
    o_ref[...] = jax.lax.dot_general(t, wom_ref[...], (((1,), (1,)), ((), ())),
                                     preferred_element_type=jnp.float32) + bom_ref[...]


def _run_finish(gathered, v1, v2, v3, i1, i2, i3, W_fc, b_fc, W_om, b_om):
    nq = v1.shape[0]
    d = W_fc.shape[0]
    return pl.pallas_call(
        _finish_body,
        out_shape=jax.ShapeDtypeStruct((nq, d), jnp.float32),
    )(gathered, v1, v2, v3, i1, i2, i3,
      W_fc, b_fc.reshape(1, d), W_om, b_om.reshape(1, d))


def kernel(new_node_features, existing_node_features, W_fc, b_fc, W_om, b_om):
    v1, v2, v3, i1, i2, i3 = _run_topk(new_node_features,
                                       existing_node_features)
    nk, d = existing_node_features.shape
    table2 = existing_node_features.reshape(nk // 2, 2 * d)
    idx = jnp.concatenate([i1, i2, i3], axis=0).reshape(1, -1) // 2
    gathered = _sc_gather(table2, idx)
    return _run_finish(gathered, v1, v2, v3, i1, i2, i3,
                       W_fc, b_fc, W_om, b_om)


# speedup vs baseline: 1.3129x; 1.3129x over previous
import jax
import jax.numpy as jnp
from jax.experimental import pallas as pl
from jax.experimental.pallas import tpu as pltpu
from jax.experimental.pallas import tpu_sc as plsc

_K_BLOCK = 4000
_K_PAD = 4096
_CHUNKS = _K_PAD // 128
_ROW_GROUP = 8
_GATHER_WINDOW = 128
_NEG = -3.0e38


def _topk_body(q_ref, k_ref, v1o, v2o, v3o, i1o, i2o, i3o,
               qn, sim_ref, v1L, v2L, v3L, c1L, c2L, c3L):
    step = pl.program_id(0)
    nsteps = pl.num_programs(0)
    nq = q_ref.shape[0]

    @pl.when(step == 0)
    def _():
        q0 = q_ref[...]
        qn[...] = 1.0 / jnp.sqrt(jnp.sum(q0 * q0, axis=1, keepdims=True))
        neg = jnp.full(v1L.shape, _NEG, jnp.float32)
        zero = jnp.zeros(c1L.shape, jnp.int32)
        v1L[...] = neg
        v2L[...] = neg
        v3L[...] = neg
        c1L[...] = zero
        c2L[...] = zero
        c3L[...] = zero
        sim_ref[:, _K_BLOCK:_K_PAD] = jnp.full((nq, _K_PAD - _K_BLOCK),
                                               _NEG, jnp.float32)

    kb = k_ref[...]
    sq = kb * kb
    ones = jnp.ones((8, kb.shape[1]), jnp.float32)
    knsq = jax.lax.dot_general(ones, sq, (((1,), (1,)), ((), ())),
                               preferred_element_type=jnp.float32,
                               precision=jax.lax.Precision.HIGHEST)[0:1]
    rkn = 1.0 / (jnp.sqrt(knsq) + 1e-30)
    s = jax.lax.dot_general(q_ref[...], kb, (((1,), (1,)), ((), ())),
                            preferred_element_type=jnp.float32)
    sim_ref[:, 0:_K_BLOCK] = s * rkn

    base_chunk = step * _CHUNKS

    for rg in range(0, nq, _ROW_GROUP):
        rows = slice(rg, rg + _ROW_GROUP)
        carry = (v1L[rows, :], v2L[rows, :], v3L[rows, :],
                 c1L[rows, :], c2L[rows, :], c3L[rows, :])

        for j in range(_CHUNKS):
            a1, a2, a3, d1, d2, d3 = carry
            x = sim_ref[rows, j * 128:(j + 1) * 128]
            cb = jnp.zeros(x.shape, jnp.int32) + (base_chunk + j)
            g1 = x > a1
            g2 = x > a2
            g3 = x > a3
            n3 = jnp.where(g2, a2, jnp.where(g3, x, a3))
            e3 = jnp.where(g2, d2, jnp.where(g3, cb, d3))
            n2 = jnp.where(g1, a1, jnp.where(g2, x, a2))
            e2 = jnp.where(g1, d1, jnp.where(g2, cb, d2))
            n1 = jnp.where(g1, x, a1)
            e1 = jnp.where(g1, cb, d1)
            carry = (n1, n2, n3, e1, e2, e3)

        v1L[rows, :], v2L[rows, :], v3L[rows, :] = carry[0], carry[1], carry[2]
        c1L[rows, :], c2L[rows, :], c3L[rows, :] = carry[3], carry[4], carry[5]

    @pl.when(step == nsteps - 1)
    def _():
        lane = jax.lax.broadcasted_iota(jnp.int32, (nq, 128), 1)

        def cand_idx(c):
            return (c >> 5) * _K_BLOCK + (c & 31) * 128 + lane

        vals = jnp.concatenate([v1L[...], v2L[...], v3L[...]], axis=1)
        idxs = jnp.concatenate([cand_idx(c1L[...]), cand_idx(c2L[...]),
                                cand_idx(c3L[...])], axis=1)
        big = jnp.int32(2 ** 30)

        def extract(vals, idxs):
            m = jnp.max(vals, axis=1, keepdims=True)
            b = jnp.min(jnp.where(vals == m, idxs, big), axis=1, keepdims=True)
            return m, b, jnp.where(idxs == b, _NEG, vals)

        m1, b1, vals = extract(vals, idxs)
        m2, b2, vals = extract(vals, idxs)
        m3, b3, _ = extract(vals, idxs)
        rq = qn[...]
        v1o[...] = m1 * rq
        v2o[...] = m2 * rq
        v3o[...] = m3 * rq
        i1o[...] = b1
        i2o[...] = b2
        i3o[...] = b3


def _run_topk(q, k):
    nq = q.shape[0]
    grid = (k.shape[0] // _K_BLOCK,)
    return pl.pallas_call(
        _topk_body,
        grid=grid,
        in_specs=[
            pl.BlockSpec((nq, q.shape[1]), lambda i: (0, 0)),
            pl.BlockSpec((_K_BLOCK, k.shape[1]), lambda i: (i, 0)),
        ],
        out_specs=[pl.BlockSpec((nq, 1), lambda i: (0, 0))] * 6,
        out_shape=[jax.ShapeDtypeStruct((nq, 1), jnp.float32)] * 3
        + [jax.ShapeDtypeStruct((nq, 1), jnp.int32)] * 3,
        scratch_shapes=[pltpu.VMEM((nq, 1), jnp.float32),
                        pltpu.VMEM((nq, _K_PAD), jnp.float32)]
        + [pltpu.VMEM((nq, 128), jnp.float32)] * 3
        + [pltpu.VMEM((nq, 128), jnp.int32)] * 3,
    )(q, k)


def _sc_gather(table, idx):
    n_idx = idx.shape[1]
    mesh = plsc.VectorSubcoreMesh(core_axis_name="core",
                                  subcore_axis_name="subcore")

    @pl.kernel(out_type=jax.ShapeDtypeStruct((n_idx, table.shape[1]),
                                             table.dtype),
               mesh=mesh)
    def _gather_kernel(x_hbm, i_hbm, o_hbm):
        def body(i_vmem, o_vmem):
            pltpu.sync_copy(x_hbm.at[i_vmem.at[0]], o_vmem)

        pltpu.emit_pipeline(
            body,
            grid=(n_idx // _GATHER_WINDOW,),
            in_specs=[pl.BlockSpec((1, _GATHER_WINDOW),
                                   index_map=lambda i: (0, i))],
            out_specs=[pl.BlockSpec((_GATHER_WINDOW, table.shape[1]),
                                    index_map=lambda i: (i, 0))],
            core_axis_name="subcore",
            dimension_semantics=(pltpu.PARALLEL,),
        )(i_hbm, o_hbm)

    return _gather_kernel(table, idx)


def _finish_body(g_ref, w1, w2, w3, i1, i2, i3,
                 wfc_ref, bfc_ref, wom_ref, bom_ref, o_ref):
    n = o_ref.shape[0]
    d = o_ref.shape[1]

    def _half(g, idx):
        par = (idx[...] % 2) == 1
        return jnp.where(par, g[:, d:2 * d], g[:, 0:d])

    g0 = _half(g_ref[0:n], i1)
    g1 = _half(g_ref[n:2 * n], i2)
    g2 = _half(g_ref[2 * n:3 * n], i3)
    a, b, c = w1[...], w2[...], w3[...]
    agg = (g0 * a + g1 * b + g2 * c) / (a + b + c)
    t = jax.lax.dot_general(agg, wfc_ref[...], (((1,), (1,)), ((), ())),
                            preferred_element_type=jnp.float32) + bfc_ref[...]
    o_ref[...] = jax.lax.dot_general(t, wom_ref[...], (((1,), (1,)), ((), ())),
                                     preferred_element_type=jnp.float32) + bom_ref[...]


def _run_finish(gathered, v1, v2, v3, i1, i2, i3, W_fc, b_fc, W_om, b_om):
    nq = v1.shape[0]
    d = W_fc.shape[0]
    return pl.pallas_call(
        _finish_body,
        out_shape=jax.ShapeDtypeStruct((nq, d), jnp.float32),
    )(gathered, v1, v2, v3, i1, i2, i3,
      W_fc, b_fc.reshape(1, d), W_om, b_om.reshape(1, d))


def kernel(new_node_features, existing_node_features, W_fc, b_fc, W_om, b_om):
    v1, v2, v3, i1, i2, i3 = _run_topk(new_node_features,
                                       existing_node_features)
    nk, d = existing_node_features.shape
    table2 = existing_node_features.reshape(nk // 2, 2 * d)
    idx = jnp.concatenate([i1, i2, i3], axis=0).reshape(1, -1) // 2
    gathered = _sc_gather(table2, idx)
    return _run_finish(gathered, v1, v2, v3, i1, i2, i3,
                       W_fc, b_fc, W_om, b_om)


# speedup vs baseline: 1.3417x; 1.0220x over previous
import jax
import jax.numpy as jnp
from jax.experimental import pallas as pl
from jax.experimental.pallas import tpu as pltpu
from jax.experimental.pallas import tpu_sc as plsc

_K_BLOCK = 5000
_K_PAD = 5120
_CHUNKS = _K_PAD // 128
_ROW_GROUP = 8
_GATHER_WINDOW = 128
_NEG = -3.0e38


def _topk_body(q_ref, k_ref, v1o, v2o, v3o, i1o, i2o, i3o,
               qn, sim_ref, v1L, v2L, v3L, c1L, c2L, c3L):
    step = pl.program_id(0)
    nsteps = pl.num_programs(0)
    nq = q_ref.shape[0]

    @pl.when(step == 0)
    def _():
        q0 = q_ref[...]
        qn[...] = 1.0 / jnp.sqrt(jnp.sum(q0 * q0, axis=1, keepdims=True))
        neg = jnp.full(v1L.shape, _NEG, jnp.float32)
        zero = jnp.zeros(c1L.shape, jnp.int32)
        v1L[...] = neg
        v2L[...] = neg
        v3L[...] = neg
        c1L[...] = zero
        c2L[...] = zero
        c3L[...] = zero
        sim_ref[:, _K_BLOCK:_K_PAD] = jnp.full((nq, _K_PAD - _K_BLOCK),
                                               _NEG, jnp.float32)

    kb = k_ref[...]
    sq = kb * kb
    ones = jnp.ones((8, kb.shape[1]), jnp.float32)
    knsq = jax.lax.dot_general(ones, sq, (((1,), (1,)), ((), ())),
                               preferred_element_type=jnp.float32,
                               precision=jax.lax.Precision.HIGHEST)[0:1]
    rkn = 1.0 / (jnp.sqrt(knsq) + 1e-30)
    s = jax.lax.dot_general(q_ref[...], kb, (((1,), (1,)), ((), ())),
                            preferred_element_type=jnp.float32)
    sim_ref[:, 0:_K_BLOCK] = s * rkn

    base_chunk = step * 64

    for rg in range(0, nq, _ROW_GROUP):
        rows = slice(rg, rg + _ROW_GROUP)
        carry = (v1L[rows, :], v2L[rows, :], v3L[rows, :],
                 c1L[rows, :], c2L[rows, :], c3L[rows, :])

        for j in range(_CHUNKS):
            a1, a2, a3, d1, d2, d3 = carry
            x = sim_ref[rows, j * 128:(j + 1) * 128]
            cb = jnp.zeros(x.shape, jnp.int32) + (base_chunk + j)
            g1 = x > a1
            g2 = x > a2
            g3 = x > a3
            n3 = jnp.where(g2, a2, jnp.where(g3, x, a3))
            e3 = jnp.where(g2, d2, jnp.where(g3, cb, d3))
            n2 = jnp.where(g1, a1, jnp.where(g2, x, a2))
            e2 = jnp.where(g1, d1, jnp.where(g2, cb, d2))
            n1 = jnp.where(g1, x, a1)
            e1 = jnp.where(g1, cb, d1)
            carry = (n1, n2, n3, e1, e2, e3)

        v1L[rows, :], v2L[rows, :], v3L[rows, :] = carry[0], carry[1], carry[2]
        c1L[rows, :], c2L[rows, :], c3L[rows, :] = carry[3], carry[4], carry[5]

    @pl.when(step == nsteps - 1)
    def _():
        lane = jax.lax.broadcasted_iota(jnp.int32, (nq, 128), 1)

        def cand_idx(c):
            return (c >> 6) * _K_BLOCK + (c & 63) * 128 + lane

        vals = jnp.concatenate([v1L[...], v2L[...], v3L[...]], axis=1)
        idxs = jnp.concatenate([cand_idx(c1L[...]), cand_idx(c2L[...]),
                                cand_idx(c3L[...])], axis=1)
        big = jnp.int32(2 ** 30)

        def extract(vals, idxs):
            m = jnp.max(vals, axis=1, keepdims=True)
            b = jnp.min(jnp.where(vals == m, idxs, big), axis=1, keepdims=True)
            return m, b, jnp.where(idxs == b, _NEG, vals)

        m1, b1, vals = extract(vals, idxs)
        m2, b2, vals = extract(vals, idxs)
        m3, b3, _ = extract(vals, idxs)
        rq = qn[...]
        v1o[...] = m1 * rq
        v2o[...] = m2 * rq
        v3o[...] = m3 * rq
        i1o[...] = b1
        i2o[...] = b2
        i3o[...] = b3


def _run_topk(q, k):
    nq = q.shape[0]
    grid = (k.shape[0] // _K_BLOCK,)
    return pl.pallas_call(
        _topk_body,
        grid=grid,
        in_specs=[
            pl.BlockSpec((nq, q.shape[1]), lambda i: (0, 0)),
            pl.BlockSpec((_K_BLOCK, k.shape[1]), lambda i: (i, 0)),
        ],
        out_specs=[pl.BlockSpec((nq, 1), lambda i: (0, 0))] * 6,
        out_shape=[jax.ShapeDtypeStruct((nq, 1), jnp.float32)] * 3
        + [jax.ShapeDtypeStruct((nq, 1), jnp.int32)] * 3,
        scratch_shapes=[pltpu.VMEM((nq, 1), jnp.float32),
                        pltpu.VMEM((nq, _K_PAD), jnp.float32)]
        + [pltpu.VMEM((nq, 128), jnp.float32)] * 3
        + [pltpu.VMEM((nq, 128), jnp.int32)] * 3,
    )(q, k)


def _sc_gather(table, idx):
    n_idx = idx.shape[1]
    mesh = plsc.VectorSubcoreMesh(core_axis_name="core",
                                  subcore_axis_name="subcore")

    @pl.kernel(out_type=jax.ShapeDtypeStruct((n_idx, table.shape[1]),
                                             table.dtype),
               mesh=mesh)
    def _gather_kernel(x_hbm, i_hbm, o_hbm):
        def body(i_vmem, o_vmem):
            pltpu.sync_copy(x_hbm.at[i_vmem.at[0]], o_vmem)

        pltpu.emit_pipeline(
            body,
            grid=(n_idx // _GATHER_WINDOW,),
            in_specs=[pl.BlockSpec((1, _GATHER_WINDOW),
                                   index_map=lambda i: (0, i))],
            out_specs=[pl.BlockSpec((_GATHER_WINDOW, table.shape[1]),
                                    index_map=lambda i: (i, 0))],
            core_axis_name="subcore",
            dimension_semantics=(pltpu.PARALLEL,),
        )(i_hbm, o_hbm)

    return _gather_kernel(table, idx)


def _finish_body(g_ref, w1, w2, w3, i1, i2, i3,
                 wfc_ref, bfc_ref, wom_ref, bom_ref, o_ref):
    n = o_ref.shape[0]
    d = o_ref.shape[1]

    def _half(g, idx):
        par = (idx[...] % 2) == 1
        return jnp.where(par, g[:, d:2 * d], g[:, 0:d])

    g0 = _half(g_ref[0:n], i1)
    g1 = _half(g_ref[n:2 * n], i2)
    g2 = _half(g_ref[2 * n:3 * n], i3)
    a, b, c = w1[...], w2[...], w3[...]
    agg = (g0 * a + g1 * b + g2 * c) / (a + b + c)
    t = jax.lax.dot_general(agg, wfc_ref[...], (((1,), (1,)), ((), ())),
                            preferred_element_type=jnp.float32) + bfc_ref[...]
    o_ref[...] = jax.lax.dot_general(t, wom_ref[...], (((1,), (1,)), ((), ())),
                                     preferred_element_type=jnp.float32) + bom_ref[...]


def _run_finish(gathered, v1, v2, v3, i1, i2, i3, W_fc, b_fc, W_om, b_om):
    nq = v1.shape[0]
    d = W_fc.shape[0]
    return pl.pallas_call(
        _finish_body,
        out_shape=jax.ShapeDtypeStruct((nq, d), jnp.float32),
    )(gathered, v1, v2, v3, i1, i2, i3,
      W_fc, b_fc.reshape(1, d), W_om, b_om.reshape(1, d))


def kernel(new_node_features, existing_node_features, W_fc, b_fc, W_om, b_om):
    v1, v2, v3, i1, i2, i3 = _run_topk(new_node_features,
                                       existing_node_features)
    nk, d = existing_node_features.shape
    table2 = existing_node_features.reshape(nk // 2, 2 * d)
    idx = jnp.concatenate([i1, i2, i3], axis=0).reshape(1, -1) // 2
    gathered = _sc_gather(table2, idx)
    return _run_finish(gathered, v1, v2, v3, i1, i2, i3,
                       W_fc, b_fc, W_om, b_om)
